# baseline (speedup 1.0000x reference)
import jax
import jax.numpy as jnp
from jax.experimental import pallas as pl

B = 16
C = 256
BC = B * C
LEVEL_HW = (4096, 1024, 256, 64)
S_TOTAL = 5440
S_TILE = 128
TS = (0, 32, 40, 42, 43)


def _kern(f0, f1, f2, f3, emb, out_ref):
    i = pl.program_id(0)
    ins = (f0, f1, f2, f3)
    for lvl in range(4):
        lo, hi = TS[lvl], TS[lvl + 1]

        @pl.when((i >= lo) & (i < hi))
        def _(lvl=lvl):
            x = ins[lvl][...]
            if LEVEL_HW[lvl] >= S_TILE:
                out_ref[...] = x.T + emb[lvl][None, :]
            else:
                out_ref[0 : LEVEL_HW[lvl], :] = x.T + emb[lvl][None, :]


def _in_spec(lvl):
    lo, n = TS[lvl], TS[lvl + 1] - TS[lvl]
    s_blk = min(S_TILE, LEVEL_HW[lvl])
    return pl.BlockSpec(
        (BC, s_blk),
        lambda i: (0, jnp.clip(i - lo, 0, n - 1)),
    )


def kernel(feats_0, feats_1, feats_2, feats_3, level_start_idx, spatial_shapes, embed_weight):
    feats = [
        f.reshape(BC, hw)
        for f, hw in zip((feats_0, feats_1, feats_2, feats_3), LEVEL_HW)
    ]
    emb_bc = jnp.tile(embed_weight, (1, B))
    out = pl.pallas_call(
        _kern,
        grid=(TS[-1],),
        in_specs=[_in_spec(l) for l in range(4)]
        + [pl.BlockSpec((4, BC), lambda i: (0, 0))],
        out_specs=pl.BlockSpec((S_TILE, BC), lambda i: (i, 0)),
        out_shape=jax.ShapeDtypeStruct((S_TOTAL, BC), jnp.float32),
    )(*feats, emb_bc)
    return out.reshape(S_TOTAL, B, C)


# speedup vs baseline: 1.0003x; 1.0003x over previous
import jax
import jax.numpy as jnp
from jax.experimental import pallas as pl
from jax.experimental.pallas import tpu as pltpu

B = 16
C = 256
BC = B * C
LEVEL_HW = (4096, 1024, 256, 64)
S_TOTAL = 5440
S_TILE = 128
TS = (0, 32, 40, 42, 43)


def _kern(f0, f1, f2, f3, emb, out_ref):
    i = pl.program_id(0)
    ins = (f0, f1, f2, f3)
    for lvl in range(4):
        lo, hi = TS[lvl], TS[lvl + 1]

        @pl.when((i >= lo) & (i < hi))
        def _(lvl=lvl):
            x = ins[lvl][...]
            if LEVEL_HW[lvl] >= S_TILE:
                out_ref[...] = x.T + emb[lvl][None, :]
            else:
                out_ref[0 : LEVEL_HW[lvl], :] = x.T + emb[lvl][None, :]


def _in_spec(lvl):
    lo, n = TS[lvl], TS[lvl + 1] - TS[lvl]
    s_blk = min(S_TILE, LEVEL_HW[lvl])
    return pl.BlockSpec(
        (BC, s_blk),
        lambda i: (0, jnp.clip(i - lo, 0, n - 1)),
    )


def kernel(feats_0, feats_1, feats_2, feats_3, level_start_idx, spatial_shapes, embed_weight):
    feats = [
        f.reshape(BC, hw)
        for f, hw in zip((feats_0, feats_1, feats_2, feats_3), LEVEL_HW)
    ]
    emb_bc = jnp.tile(embed_weight, (1, B))
    out = pl.pallas_call(
        _kern,
        grid=(TS[-1],),
        in_specs=[_in_spec(l) for l in range(4)]
        + [pl.BlockSpec((4, BC), lambda i: (0, 0))],
        out_specs=pl.BlockSpec((S_TILE, BC), lambda i: (i, 0)),
        out_shape=jax.ShapeDtypeStruct((S_TOTAL, BC), jnp.float32),
        compiler_params=pltpu.CompilerParams(
            dimension_semantics=("parallel",),
        ),
    )(*feats, emb_bc)
    return out.reshape(S_TOTAL, B, C)


# speedup vs baseline: 1.0330x; 1.0327x over previous
import jax
import jax.numpy as jnp
from jax.experimental import pallas as pl
from jax.experimental.pallas import tpu as pltpu

B = 16
C = 256
BC = B * C
LEVEL_HW = (4096, 1024, 256, 64)
S_TOTAL = 5440
S_TILE = 256
TS = (0, 16, 20, 21, 22)


def _kern(f0, f1, f2, f3, emb, out_ref):
    i = pl.program_id(0)
    ins = (f0, f1, f2, f3)
    for lvl in range(4):
        lo, hi = TS[lvl], TS[lvl + 1]

        @pl.when((i >= lo) & (i < hi))
        def _(lvl=lvl):
            x = ins[lvl][...]
            if LEVEL_HW[lvl] >= S_TILE:
                out_ref[...] = x.T + emb[lvl][None, :]
            else:
                out_ref[0 : LEVEL_HW[lvl], :] = x.T + emb[lvl][None, :]


def _in_spec(lvl):
    lo, n = TS[lvl], TS[lvl + 1] - TS[lvl]
    s_blk = min(S_TILE, LEVEL_HW[lvl])
    return pl.BlockSpec(
        (BC, s_blk),
        lambda i: (0, jnp.clip(i - lo, 0, n - 1)),
    )


def kernel(feats_0, feats_1, feats_2, feats_3, level_start_idx, spatial_shapes, embed_weight):
    feats = [
        f.reshape(BC, hw)
        for f, hw in zip((feats_0, feats_1, feats_2, feats_3), LEVEL_HW)
    ]
    emb_bc = jnp.tile(embed_weight, (1, B))
    out = pl.pallas_call(
        _kern,
        grid=(TS[-1],),
        in_specs=[_in_spec(l) for l in range(4)]
        + [pl.BlockSpec((4, BC), lambda i: (0, 0))],
        out_specs=pl.BlockSpec((S_TILE, BC), lambda i: (i, 0)),
        out_shape=jax.ShapeDtypeStruct((S_TOTAL, BC), jnp.float32),
        compiler_params=pltpu.CompilerParams(
            dimension_semantics=("parallel",),
        ),
    )(*feats, emb_bc)
    return out.reshape(S_TOTAL, B, C)
